# baseline (speedup 1.0000x reference)
import jax
import jax.numpy as jnp
from jax import lax
from jax.experimental import pallas as pl
from jax.experimental.pallas import tpu as pltpu
from jax.experimental.pallas import tpu_sc as plsc

_B, _N, _T = 64, 325, 12
_SPACE_D, _DAY_D, _WEEK_D = 64, 32, 32
_DAY_V, _WEEK_V = 288, 7
_OUT_D = 1 + _SPACE_D + _DAY_D + _WEEK_D
_R = _B * _N * _T
_CP = 20
_CR = _CP * _T
_NG = _CR // 16
_CPB = -(-_N // _CP)
_NCHUNK = 2 * _CPB
_LAST_N0 = _N - _CP


def _sc_body(f_hbm, di_hbm, wi_hbm, space_hbm, day_hbm, week_hbm, out_hbm,
             f_v, di_v, wi_v, space_v, day_v, week_v, obuf_v):
    wid = lax.axis_index("s") * 2 + lax.axis_index("c")

    pltpu.sync_copy(space_hbm, space_v)
    pltpu.sync_copy(day_hbm, day_v)
    pltpu.sync_copy(week_hbm, week_v)

    lanes = lax.iota(jnp.int32, 16)

    def chunk(ci, carry):
        b = wid * 2 + ci // _CPB
        n0 = jnp.minimum((ci % _CPB) * _CP, _LAST_N0)
        pltpu.sync_copy(f_hbm.at[b, pl.ds(n0, _CP), :], f_v)
        pltpu.sync_copy(di_hbm.at[b, pl.ds(n0, _CP), :], di_v)
        pltpu.sync_copy(wi_hbm.at[b, pl.ds(n0, _CP), :], wi_v)

        dv, wv = [], []
        for g in range(_NG):
            base = g * 16
            r = base + lanes
            n_i = r // _T
            t_i = r - n_i * _T
            zeros = lanes * 0
            fvec = plsc.load_gather(f_v, [n_i, t_i])
            dvec = plsc.load_gather(di_v, [n_i, t_i])
            wvec = plsc.load_gather(wi_v, [n_i, t_i])
            plsc.store_scatter(obuf_v, [r, zeros], fvec)
            w31 = plsc.load_gather(week_v, [wvec * _WEEK_D + 31])
            plsc.store_scatter(obuf_v, [r, zeros + 128], w31)
            dv.append(dvec * _DAY_D)
            wv.append(wvec * _WEEK_D)

        for p in range(_CP):
            sb = (n0 + p) * _SPACE_D
            for t in range(_T):
                rr = p * _T + t
                g, l = rr // 16, rr % 16
                db = dv[g][l]
                wb = wv[g][l]
                for k in range(4):
                    obuf_v[rr, pl.ds(1 + 16 * k, 16)] = space_v[pl.ds(sb + 16 * k, 16)]
                for k in range(2):
                    obuf_v[rr, pl.ds(65 + 16 * k, 16)] = day_v[pl.ds(db + 16 * k, 16)]
                obuf_v[rr, pl.ds(97, 16)] = week_v[pl.ds(wb, 16)]
                obuf_v[rr, pl.ds(112, 16)] = week_v[pl.ds(wb + 15, 16)]

        rbase = (b * _N + n0) * _T
        pltpu.sync_copy(obuf_v, out_hbm.at[pl.ds(rbase, _CR), :])
        return carry

    lax.fori_loop(0, _NCHUNK, chunk, 0)


def kernel(X, spaceInfo, dayInfo, weekInfo):
    pad = ((0, 0), (0, 3), (0, 128 - _T))
    featx = jnp.pad(X[..., 0], pad)
    dayi = jnp.pad(X[..., 1].astype(jnp.int32), pad)
    weeki = jnp.pad(X[..., 2].astype(jnp.int32), pad)
    mesh = plsc.VectorSubcoreMesh(core_axis_name="c", subcore_axis_name="s")
    out = pl.kernel(
        _sc_body,
        mesh=mesh,
        compiler_params=pltpu.CompilerParams(
            needs_layout_passes=False, use_tc_tiling_on_sc=False),
        out_type=jax.ShapeDtypeStruct((_R, _OUT_D), jnp.float32),
        scratch_types=[
            pltpu.VMEM((_CP, 128), jnp.float32),
            pltpu.VMEM((_CP, 128), jnp.int32),
            pltpu.VMEM((_CP, 128), jnp.int32),
            pltpu.VMEM((_N * _SPACE_D,), jnp.float32),
            pltpu.VMEM((_DAY_V * _DAY_D,), jnp.float32),
            pltpu.VMEM((_WEEK_V * _WEEK_D,), jnp.float32),
            pltpu.VMEM((_CR, _OUT_D), jnp.float32),
        ],
    )(featx, dayi, weeki, spaceInfo.reshape(-1),
      dayInfo.reshape(-1), weekInfo.reshape(-1))
    return out.reshape(_B, _N, _T, _OUT_D)


# speedup vs baseline: 1.3200x; 1.3200x over previous
import jax
import jax.numpy as jnp
from jax import lax
from jax.experimental import pallas as pl
from jax.experimental.pallas import tpu as pltpu
from jax.experimental.pallas import tpu_sc as plsc

_B, _N, _T = 64, 325, 12
_SPACE_D, _DAY_D, _WEEK_D = 64, 32, 32
_DAY_V, _WEEK_V = 288, 7
_OUT_D = 1 + _SPACE_D + _DAY_D + _WEEK_D
_R = _B * _N * _T
_NW = 32
_RPW = _R // _NW
_C = 480
_NCHUNK = -(-_RPW // _C)
_LAST_OFF = _RPW - _C
_G = _C // 16


def _body(x_hbm, space_hbm, day_hbm, week_hbm, out_hbm,
          x_v, space_v, day_v, week_v, out_v):
    wid = lax.axis_index("s") * 2 + lax.axis_index("c")
    wbase = wid * _RPW

    pltpu.sync_copy(space_hbm, space_v)
    pltpu.sync_copy(day_hbm, day_v)
    pltpu.sync_copy(week_hbm, week_v)

    lanes = lax.iota(jnp.int32, 16)
    lanes3 = lanes * 3
    lanes_out = lanes * _OUT_D

    def chunk(ci, carry):
        off = jnp.minimum(ci * _C, _LAST_OFF)
        cbase = wbase + off
        pltpu.sync_copy(x_hbm.at[pl.ds(cbase * 3, _C * 3)], x_v)

        def group(g, carry2):
            b = g * 16
            idx0 = b * 3 + lanes3
            fvec = plsc.load_gather(x_v, [idx0])
            dvec = plsc.load_gather(x_v, [idx0 + 1]).astype(jnp.int32)
            wvec = plsc.load_gather(x_v, [idx0 + 2]).astype(jnp.int32)
            plsc.store_scatter(out_v, [b * _OUT_D + lanes_out], fvec)
            for j in range(16):
                o = (b + j) * _OUT_D
                sb = (((cbase + b + j) // _T) % _N) * _SPACE_D
                db = dvec[j] * _DAY_D
                wb = wvec[j] * _WEEK_D
                for k in range(4):
                    out_v[pl.ds(o + 1 + 16 * k, 16)] = space_v[pl.ds(sb + 16 * k, 16)]
                for k in range(2):
                    out_v[pl.ds(o + 65 + 16 * k, 16)] = day_v[pl.ds(db + 16 * k, 16)]
                for k in range(2):
                    out_v[pl.ds(o + 97 + 16 * k, 16)] = week_v[pl.ds(wb + 16 * k, 16)]
            return carry2

        lax.fori_loop(0, _G, group, 0)
        pltpu.sync_copy(out_v, out_hbm.at[pl.ds(cbase * _OUT_D, _C * _OUT_D)])
        return carry

    lax.fori_loop(0, _NCHUNK, chunk, 0)


def kernel(X, spaceInfo, dayInfo, weekInfo):
    x_flat = X.reshape(_R * 3)
    mesh = plsc.VectorSubcoreMesh(core_axis_name="c", subcore_axis_name="s")
    out = pl.kernel(
        _body,
        mesh=mesh,
        compiler_params=pltpu.CompilerParams(needs_layout_passes=False),
        out_type=jax.ShapeDtypeStruct((_R * _OUT_D,), jnp.float32),
        scratch_types=[
            pltpu.VMEM((_C * 3,), jnp.float32),
            pltpu.VMEM((_N * _SPACE_D,), jnp.float32),
            pltpu.VMEM((_DAY_V * _DAY_D,), jnp.float32),
            pltpu.VMEM((_WEEK_V * _WEEK_D,), jnp.float32),
            pltpu.VMEM((_C * _OUT_D,), jnp.float32),
        ],
    )(x_flat, spaceInfo.reshape(-1), dayInfo.reshape(-1), weekInfo.reshape(-1))
    return out.reshape(_B, _N, _T, _OUT_D)
